# Initial kernel scaffold; baseline (speedup 1.0000x reference)
import functools

import jax
import jax.numpy as jnp
from jax import lax
from jax.experimental import pallas as pl
from jax.experimental.pallas import tpu as pltpu
from jax.experimental.pallas import tpu_sc as plsc

_NUM_SC_CORES = 2
_NUM_SC_SUBCORES = 16
_NUM_WORKERS = _NUM_SC_CORES * _NUM_SC_SUBCORES


def _sc_gather(table, idx, emb):
    n_idx = idx.shape[0]
    b_per_w = n_idx // _NUM_WORKERS
    assert n_idx % _NUM_WORKERS == 0 and b_per_w % 8 == 0

    mesh = plsc.VectorSubcoreMesh(core_axis_name="c", subcore_axis_name="s")

    @functools.partial(
        pl.kernel,
        mesh=mesh,
        out_type=jax.ShapeDtypeStruct((n_idx, emb), table.dtype),
        scratch_types=[
            pltpu.VMEM((b_per_w,), jnp.int32),
            pltpu.VMEM((b_per_w, emb), table.dtype),
            pltpu.SemaphoreType.DMA,
        ],
    )
    def gather_kernel(table_hbm, idx_hbm, out_hbm, idx_v, rows_v, sem):
        wid = lax.axis_index("s") * _NUM_SC_CORES + lax.axis_index("c")
        base = wid * b_per_w
        pltpu.sync_copy(idx_hbm.at[pl.ds(base, b_per_w)], idx_v)
        pltpu.async_copy(table_hbm.at[idx_v], rows_v, sem).wait()
        pltpu.sync_copy(rows_v, out_hbm.at[pl.ds(base, b_per_w)])

    return gather_kernel(table, idx)


def _mlp_body(c_ref, w1_ref, b1_ref, w2_ref, b2_ref, o_ref):
    c = c_ref[...].astype(jnp.bfloat16)
    w1 = w1_ref[...].astype(jnp.bfloat16)
    h = lax.dot_general(c, w1, (((1,), (0,)), ((), ())),
                        preferred_element_type=jnp.float32)
    h = jnp.maximum(h + b1_ref[...], 0.0).astype(jnp.bfloat16)
    w2 = w2_ref[...].astype(jnp.bfloat16)
    o = lax.dot_general(h, w2, (((1,), (0,)), ((), ())),
                        preferred_element_type=jnp.float32)
    o_ref[...] = o + b2_ref[...]


def _mlp(combined, w1, b1, w2, b2, interpret=False):
    b, k = combined.shape
    hid = w1.shape[1]
    bm = 1024
    return pl.pallas_call(
        _mlp_body,
        grid=(b // bm,),
        in_specs=[
            pl.BlockSpec((bm, k), lambda i: (i, 0)),
            pl.BlockSpec((k, hid), lambda i: (0, 0)),
            pl.BlockSpec((1, hid), lambda i: (0, 0)),
            pl.BlockSpec((hid, hid), lambda i: (0, 0)),
            pl.BlockSpec((1, hid), lambda i: (0, 0)),
        ],
        out_specs=pl.BlockSpec((bm, hid), lambda i: (i, 0)),
        out_shape=jax.ShapeDtypeStruct((b, hid), jnp.float32),
        interpret=interpret,
    )(combined, w1, b1.reshape(1, hid), w2, b2.reshape(1, hid))


def kernel(age, gender, disability, age_table, gender_table, disability_table,
           W1, b1, W2, b2):
    b = age.shape[0]
    emb = age_table.shape[1]
    n_age = age_table.shape[0]
    n_gender = gender_table.shape[0]

    table = jnp.concatenate([age_table, gender_table, disability_table], axis=0)
    idx = jnp.stack(
        [age.astype(jnp.int32),
         gender.astype(jnp.int32) + n_age,
         disability.astype(jnp.int32) + n_age + n_gender],
        axis=1,
    ).reshape(-1)

    rows = _sc_gather(table, idx, emb)
    combined = rows.reshape(b, 3 * emb)
    return _mlp(combined, W1, b1, W2, b2)


# speedup vs baseline: 1.6995x; 1.6995x over previous
import functools

import jax
import jax.numpy as jnp
from jax import lax
from jax.experimental import pallas as pl
from jax.experimental.pallas import tpu as pltpu
from jax.experimental.pallas import tpu_sc as plsc

_NUM_SC_CORES = 2
_NUM_SC_SUBCORES = 16
_NUM_WORKERS = _NUM_SC_CORES * _NUM_SC_SUBCORES


_GCHUNK = 128


def _sc_gather(table, idx, width):
    n_idx = idx.shape[0]
    b_per_w = n_idx // _NUM_WORKERS
    assert n_idx % _NUM_WORKERS == 0 and b_per_w % _GCHUNK == 0
    n_chunks = b_per_w // _GCHUNK

    mesh = plsc.VectorSubcoreMesh(core_axis_name="c", subcore_axis_name="s")

    @functools.partial(
        pl.kernel,
        mesh=mesh,
        out_type=jax.ShapeDtypeStruct((n_idx, width), table.dtype),
        scratch_types=[
            pltpu.VMEM((b_per_w,), jnp.int32),
            pltpu.VMEM((_GCHUNK, width), table.dtype),
            pltpu.SemaphoreType.DMA,
        ],
    )
    def gather_kernel(table_hbm, idx_hbm, out_hbm, idx_v, rows_v, sem):
        wid = lax.axis_index("s") * _NUM_SC_CORES + lax.axis_index("c")
        base = wid * b_per_w
        pltpu.sync_copy(idx_hbm.at[pl.ds(base, b_per_w)], idx_v)

        @pl.loop(0, n_chunks)
        def _(c):
            off = c * _GCHUNK
            pltpu.async_copy(
                table_hbm.at[idx_v.at[pl.ds(off, _GCHUNK)]], rows_v, sem
            ).wait()
            pltpu.sync_copy(rows_v, out_hbm.at[pl.ds(base + off, _GCHUNK)])

    return gather_kernel(table, idx)


def _mlp_body(c_ref, w1_ref, b1_ref, w2_ref, b2_ref, o_ref):
    c = c_ref[...].astype(jnp.bfloat16)
    w1 = w1_ref[...].astype(jnp.bfloat16)
    h = lax.dot_general(c, w1, (((1,), (0,)), ((), ())),
                        preferred_element_type=jnp.float32)
    h = jnp.maximum(h + b1_ref[...], 0.0).astype(jnp.bfloat16)
    w2 = w2_ref[...].astype(jnp.bfloat16)
    o = lax.dot_general(h, w2, (((1,), (0,)), ((), ())),
                        preferred_element_type=jnp.float32)
    o_ref[...] = o + b2_ref[...]


def _mlp(combined, w1, b1, w2, b2, interpret=False):
    b, k = combined.shape
    hid = w1.shape[1]
    bm = 1024
    return pl.pallas_call(
        _mlp_body,
        grid=(b // bm,),
        in_specs=[
            pl.BlockSpec((bm, k), lambda i: (i, 0)),
            pl.BlockSpec((k, hid), lambda i: (0, 0)),
            pl.BlockSpec((1, hid), lambda i: (0, 0)),
            pl.BlockSpec((hid, hid), lambda i: (0, 0)),
            pl.BlockSpec((1, hid), lambda i: (0, 0)),
        ],
        out_specs=pl.BlockSpec((bm, hid), lambda i: (i, 0)),
        out_shape=jax.ShapeDtypeStruct((b, hid), jnp.float32),
        interpret=interpret,
    )(combined, w1, b1.reshape(1, hid), w2, b2.reshape(1, hid))


def kernel(age, gender, disability, age_table, gender_table, disability_table,
           W1, b1, W2, b2):
    b = age.shape[0]
    emb = age_table.shape[1]
    n_age = age_table.shape[0]
    n_gender = gender_table.shape[0]
    n_dis = disability_table.shape[0]
    width = 2 * emb

    age_padded = jnp.pad(age_table, ((0, 0), (0, width - emb)))
    pair_table = jnp.concatenate(
        [jnp.broadcast_to(gender_table[:, None, :], (n_gender, n_dis, emb)),
         jnp.broadcast_to(disability_table[None, :, :], (n_gender, n_dis, emb))],
        axis=-1,
    ).reshape(n_gender * n_dis, width)
    table = jnp.concatenate([age_padded, pair_table], axis=0)
    idx = jnp.stack(
        [age.astype(jnp.int32),
         n_age + gender.astype(jnp.int32) * n_dis + disability.astype(jnp.int32)],
        axis=1,
    ).reshape(-1)

    rows = _sc_gather(table, idx, width)
    combined = rows.reshape(b, 2 * width)

    hid = W1.shape[1]
    w1p = jnp.concatenate(
        [W1[:emb], jnp.zeros((width - emb, hid), W1.dtype), W1[emb:]], axis=0)
    return _mlp(combined, w1p, b1, W2, b2)


# speedup vs baseline: 1.8549x; 1.0914x over previous
import functools

import jax
import jax.numpy as jnp
from jax import lax
from jax.experimental import pallas as pl
from jax.experimental.pallas import tpu as pltpu
from jax.experimental.pallas import tpu_sc as plsc

_NUM_SC_CORES = 2
_NUM_SC_SUBCORES = 16
_NUM_WORKERS = _NUM_SC_CORES * _NUM_SC_SUBCORES

_GCHUNK = 128
_VREG = 16


def _sc_gather_planes(table, age, gender, disability, width, pair_base,
                      n_dis):
    b = age.shape[0]
    b_per_w = b // _NUM_WORKERS
    assert b % _NUM_WORKERS == 0 and b_per_w % _GCHUNK == 0
    n_chunks = b_per_w // _GCHUNK

    mesh = plsc.VectorSubcoreMesh(core_axis_name="c", subcore_axis_name="s")

    @functools.partial(
        pl.kernel,
        mesh=mesh,
        out_type=jax.ShapeDtypeStruct((2 * b, width), table.dtype),
        scratch_types=[
            pltpu.VMEM((b_per_w,), jnp.int32),
            pltpu.VMEM((b_per_w,), jnp.int32),
            pltpu.VMEM((b_per_w,), jnp.int32),
            pltpu.VMEM((_GCHUNK, width), table.dtype),
            pltpu.VMEM((_GCHUNK, width), table.dtype),
            pltpu.SemaphoreType.DMA,
            pltpu.SemaphoreType.DMA,
        ],
    )
    def gather_kernel(table_hbm, age_hbm, gender_hbm, dis_hbm, out_hbm,
                      idxa_v, idxp_v, idxd_v, rows0_v, rows1_v, sem0, sem1):
        wid = lax.axis_index("s") * _NUM_SC_CORES + lax.axis_index("c")
        base = wid * b_per_w

        pltpu.sync_copy(age_hbm.at[pl.ds(base, b_per_w)], idxa_v)
        pltpu.sync_copy(gender_hbm.at[pl.ds(base, b_per_w)], idxp_v)
        pltpu.sync_copy(dis_hbm.at[pl.ds(base, b_per_w)], idxd_v)

        @pl.loop(0, b_per_w, step=_VREG)
        def _(i):
            slc = pl.ds(i, _VREG)
            idxp_v[slc] = idxp_v[slc] * n_dis + idxd_v[slc] + pair_base

        @pl.loop(0, n_chunks)
        def _(c):
            off = c * _GCHUNK
            pltpu.async_copy(
                table_hbm.at[idxa_v.at[pl.ds(off, _GCHUNK)]], rows0_v, sem0
            ).wait()
            pltpu.sync_copy(rows0_v, out_hbm.at[pl.ds(base + off, _GCHUNK)])

        @pl.loop(0, n_chunks)
        def _(c):
            off = c * _GCHUNK
            pltpu.async_copy(
                table_hbm.at[idxp_v.at[pl.ds(off, _GCHUNK)]], rows1_v, sem1
            ).wait()
            pltpu.sync_copy(rows1_v,
                            out_hbm.at[pl.ds(b + base + off, _GCHUNK)])

    return gather_kernel(table, age, gender, disability)


def _mlp_body(a_ref, p_ref, w1_ref, b1_ref, w2_ref, b2_ref, o_ref):
    width = a_ref.shape[1]
    emb = width // 2
    a = a_ref[...].astype(jnp.bfloat16)
    p = p_ref[...].astype(jnp.bfloat16)
    w1a = w1_ref[0:width, :].astype(jnp.bfloat16)
    w1p = w1_ref[emb:emb + width, :].astype(jnp.bfloat16)
    dn = (((1,), (0,)), ((), ()))
    h = (lax.dot_general(a, w1a, dn, preferred_element_type=jnp.float32)
         + lax.dot_general(p, w1p, dn, preferred_element_type=jnp.float32))
    h = jnp.maximum(h + b1_ref[...], 0.0).astype(jnp.bfloat16)
    w2 = w2_ref[...].astype(jnp.bfloat16)
    o = lax.dot_general(h, w2, dn, preferred_element_type=jnp.float32)
    o_ref[...] = o + b2_ref[...]


def _mlp(rows, w1, b1, w2, b2, interpret=False):
    two_b, width = rows.shape
    b = two_b // 2
    k, hid = w1.shape
    bm = 1024
    nb = b // bm
    return pl.pallas_call(
        _mlp_body,
        grid=(nb,),
        in_specs=[
            pl.BlockSpec((bm, width), lambda i: (i, 0)),
            pl.BlockSpec((bm, width), lambda i, nb=nb: (i + nb, 0)),
            pl.BlockSpec((k, hid), lambda i: (0, 0)),
            pl.BlockSpec((1, hid), lambda i: (0, 0)),
            pl.BlockSpec((hid, hid), lambda i: (0, 0)),
            pl.BlockSpec((1, hid), lambda i: (0, 0)),
        ],
        out_specs=pl.BlockSpec((bm, hid), lambda i: (i, 0)),
        out_shape=jax.ShapeDtypeStruct((b, hid), jnp.float32),
        interpret=interpret,
    )(rows, rows, w1, b1.reshape(1, hid), w2, b2.reshape(1, hid))


def kernel(age, gender, disability, age_table, gender_table, disability_table,
           W1, b1, W2, b2):
    emb = age_table.shape[1]
    n_age = age_table.shape[0]
    n_gender = gender_table.shape[0]
    n_dis = disability_table.shape[0]
    width = 2 * emb

    age_padded = jnp.pad(age_table, ((0, 0), (0, width - emb)))
    pair_table = jnp.concatenate(
        [jnp.broadcast_to(gender_table[:, None, :], (n_gender, n_dis, emb)),
         jnp.broadcast_to(disability_table[None, :, :], (n_gender, n_dis, emb))],
        axis=-1,
    ).reshape(n_gender * n_dis, width)
    table = jnp.concatenate([age_padded, pair_table], axis=0)

    rows = _sc_gather_planes(table, age.astype(jnp.int32),
                             gender.astype(jnp.int32),
                             disability.astype(jnp.int32), width,
                             pair_base=n_age, n_dis=n_dis)
    return _mlp(rows, W1, b1, W2, b2)


# speedup vs baseline: 1.9839x; 1.0696x over previous
import functools

import jax
import jax.numpy as jnp
from jax import lax
from jax.experimental import pallas as pl
from jax.experimental.pallas import tpu as pltpu
from jax.experimental.pallas import tpu_sc as plsc

_NUM_SC_CORES = 2
_NUM_SC_SUBCORES = 16
_NUM_WORKERS = _NUM_SC_CORES * _NUM_SC_SUBCORES

_GCHUNK = 128
_VREG = 16


def _sc_gather_combined(table, age, gender, disability, width, pair_base,
                        n_dis):
    b = age.shape[0]
    b_per_w = b // _NUM_WORKERS
    assert b % _NUM_WORKERS == 0 and b_per_w % _GCHUNK == 0
    n_chunks = b_per_w // _GCHUNK

    mesh = plsc.VectorSubcoreMesh(core_axis_name="c", subcore_axis_name="s")

    @functools.partial(
        pl.kernel,
        mesh=mesh,
        out_type=jax.ShapeDtypeStruct((b, 2 * width), table.dtype),
        scratch_types=[
            pltpu.VMEM((b_per_w,), jnp.int32),
            pltpu.VMEM((b_per_w,), jnp.int32),
            pltpu.VMEM((b_per_w,), jnp.int32),
            pltpu.VMEM((_GCHUNK, width), table.dtype),
            pltpu.VMEM((_GCHUNK, width), table.dtype),
            pltpu.SemaphoreType.DMA,
            pltpu.SemaphoreType.DMA,
            pltpu.SemaphoreType.DMA,
            pltpu.SemaphoreType.DMA,
        ],
    )
    def gather_kernel(table_hbm, age_hbm, gender_hbm, dis_hbm, out_hbm,
                      idxa_v, idxp_v, idxd_v, buf0, buf1, g0, g1, w0, w1):
        wid = lax.axis_index("s") * _NUM_SC_CORES + lax.axis_index("c")
        base = wid * b_per_w

        pltpu.sync_copy(age_hbm.at[pl.ds(base, b_per_w)], idxa_v)
        pltpu.sync_copy(gender_hbm.at[pl.ds(base, b_per_w)], idxp_v)
        pltpu.sync_copy(dis_hbm.at[pl.ds(base, b_per_w)], idxd_v)

        @pl.loop(0, b_per_w, step=_VREG)
        def _(i):
            slc = pl.ds(i, _VREG)
            idxp_v[slc] = idxp_v[slc] * n_dis + idxd_v[slc] + pair_base

        jobs = ([(idxa_v, 0, c) for c in range(n_chunks)]
                + [(idxp_v, width, c) for c in range(n_chunks)])
        bufs = (buf0, buf1)
        gsems = (g0, g1)
        wsems = (w0, w1)

        def start_gather(j):
            idx_v, _, c = jobs[j]
            return pltpu.async_copy(
                table_hbm.at[idx_v.at[pl.ds(c * _GCHUNK, _GCHUNK)]],
                bufs[j % 2], gsems[j % 2])

        def start_writeout(j):
            _, col, c = jobs[j]
            return pltpu.async_copy(
                bufs[j % 2],
                out_hbm.at[pl.ds(base + c * _GCHUNK, _GCHUNK),
                           pl.ds(col, width)],
                wsems[j % 2])

        n_jobs = len(jobs)
        gathers = [None] * n_jobs
        writes = [None] * n_jobs
        gathers[0] = start_gather(0)
        for j in range(n_jobs):
            gathers[j].wait()
            if j + 1 < n_jobs:
                if j >= 1:
                    writes[j - 1].wait()
                gathers[j + 1] = start_gather(j + 1)
            writes[j] = start_writeout(j)
        writes[n_jobs - 2].wait()
        writes[n_jobs - 1].wait()

    return gather_kernel(table, age, gender, disability)


def _mlp_body(c_ref, w1_ref, b1_ref, w2_ref, b2_ref, o_ref):
    c = c_ref[...].astype(jnp.bfloat16)
    w1 = w1_ref[...].astype(jnp.bfloat16)
    dn = (((1,), (0,)), ((), ()))
    h = lax.dot_general(c, w1, dn, preferred_element_type=jnp.float32)
    h = jnp.maximum(h + b1_ref[...], 0.0).astype(jnp.bfloat16)
    w2 = w2_ref[...].astype(jnp.bfloat16)
    o = lax.dot_general(h, w2, dn, preferred_element_type=jnp.float32)
    o_ref[...] = o + b2_ref[...]


def _mlp(combined, w1, b1, w2, b2, interpret=False):
    b, k = combined.shape
    hid = w1.shape[1]
    bm = 1024
    return pl.pallas_call(
        _mlp_body,
        grid=(b // bm,),
        in_specs=[
            pl.BlockSpec((bm, k), lambda i: (i, 0)),
            pl.BlockSpec((k, hid), lambda i: (0, 0)),
            pl.BlockSpec((1, hid), lambda i: (0, 0)),
            pl.BlockSpec((hid, hid), lambda i: (0, 0)),
            pl.BlockSpec((1, hid), lambda i: (0, 0)),
        ],
        out_specs=pl.BlockSpec((bm, hid), lambda i: (i, 0)),
        out_shape=jax.ShapeDtypeStruct((b, hid), jnp.float32),
        interpret=interpret,
    )(combined, w1, b1.reshape(1, hid), w2, b2.reshape(1, hid))


def kernel(age, gender, disability, age_table, gender_table, disability_table,
           W1, b1, W2, b2):
    emb = age_table.shape[1]
    n_age = age_table.shape[0]
    n_gender = gender_table.shape[0]
    n_dis = disability_table.shape[0]
    width = 2 * emb

    age_padded = jnp.pad(age_table, ((0, 0), (0, width - emb)))
    pair_table = jnp.concatenate(
        [jnp.broadcast_to(gender_table[:, None, :], (n_gender, n_dis, emb)),
         jnp.broadcast_to(disability_table[None, :, :], (n_gender, n_dis, emb))],
        axis=-1,
    ).reshape(n_gender * n_dis, width)
    table = jnp.concatenate([age_padded, pair_table], axis=0)

    combined = _sc_gather_combined(table, age.astype(jnp.int32),
                                   gender.astype(jnp.int32),
                                   disability.astype(jnp.int32), width,
                                   pair_base=n_age, n_dis=n_dis)

    hid = W1.shape[1]
    w1p = jnp.concatenate(
        [W1[:emb], jnp.zeros((width - emb, hid), W1.dtype), W1[emb:]], axis=0)
    return _mlp(combined, w1p, b1, W2, b2)
